# baseline (speedup 1.0000x reference)
import jax
import jax.numpy as jnp
from jax import lax
from jax.experimental import pallas as pl
from jax.experimental.pallas import tpu as pltpu
from jax.experimental.pallas import tpu_sc as plsc

BATCH = 4096
SEQ = 200
EMBED = 64
EPAD = 128
NUM_TARGETS = 3811

NC, NS = 2, 16
NW = NC * NS
ROWS_PER_W = BATCH // NW
CB = 4
NBLK = ROWS_PER_W // CB
LANES = 16
CCH = EMBED // LANES
GCH = SEQ // 2


def _pooling_kernel(idx_hbm, table_hbm, out_hbm, idx_v, rows_v, out_v, sem):
    wid = lax.axis_index("s") * NC + lax.axis_index("c")
    base = wid * ROWS_PER_W

    def block(blk, _):
        b0 = base + blk * CB
        pltpu.sync_copy(idx_hbm.at[pl.ds(b0, CB)], idx_v)
        cps = []
        for i in range(CB):
            for off, ln in ((0, 128), (128, 72)):
                cps.append(
                    pltpu.async_copy(
                        table_hbm.at[idx_v.at[i, pl.ds(off, ln)]],
                        rows_v.at[i, pl.ds(off, ln)],
                        sem,
                    )
                )
        for cp in cps:
            cp.wait()

        def acc_body(r, acc):
            new = []
            for i in range(CB):
                for c in range(CCH):
                    v = acc[i * CCH + c]
                    v = v + rows_v[i, r, pl.ds(LANES * c, LANES)]
                    v = v + rows_v[i, r + GCH, pl.ds(LANES * c, LANES)]
                    new.append(v)
            return tuple(new)

        zeros = tuple(
            jnp.zeros((LANES,), jnp.float32) for _ in range(CB * CCH)
        )
        acc = lax.fori_loop(0, GCH, acc_body, zeros)
        scale = jnp.float32(1.0 / SEQ)
        zv = jnp.zeros((LANES,), jnp.float32)
        for i in range(CB):
            for c in range(CCH):
                out_v[i, pl.ds(LANES * c, LANES)] = acc[i * CCH + c] * scale
            for c in range(CCH, EPAD // LANES):
                out_v[i, pl.ds(LANES * c, LANES)] = zv
        pltpu.sync_copy(out_v, out_hbm.at[pl.ds(b0, CB)])
        return ()

    lax.fori_loop(0, NBLK, block, ())


@jax.jit
def _pooled_sc(idx, table128):
    mesh = plsc.VectorSubcoreMesh(
        core_axis_name="c", subcore_axis_name="s", num_cores=NC, num_subcores=NS
    )
    return pl.kernel(
        _pooling_kernel,
        out_type=jax.ShapeDtypeStruct((BATCH, EPAD), jnp.float32),
        mesh=mesh,
        scratch_types=[
            pltpu.VMEM((CB, SEQ), jnp.int32),
            pltpu.VMEM((CB, SEQ, EPAD), jnp.float32),
            pltpu.VMEM((CB, EPAD), jnp.float32),
            pltpu.SemaphoreType.DMA,
        ],
    )(idx, table128)


def _dense_kernel(x_ref, w_ref, b_ref, o_ref):
    y = jnp.dot(x_ref[...], w_ref[...], preferred_element_type=jnp.float32)
    o_ref[...] = jax.nn.sigmoid(y + b_ref[...])


BM = 512


@jax.jit
def _dense_tc(pooled, w2, b2):
    return pl.pallas_call(
        _dense_kernel,
        grid=(BATCH // BM,),
        in_specs=[
            pl.BlockSpec((BM, EPAD), lambda i: (i, 0)),
            pl.BlockSpec((EPAD, NUM_TARGETS), lambda i: (0, 0)),
            pl.BlockSpec((1, NUM_TARGETS), lambda i: (0, 0)),
        ],
        out_specs=pl.BlockSpec((BM, NUM_TARGETS), lambda i: (i, 0)),
        out_shape=jax.ShapeDtypeStruct((BATCH, NUM_TARGETS), jnp.float32),
    )(pooled, w2, b2)


def kernel(inputs, table, W, b):
    table128 = jnp.pad(table, ((0, 0), (0, EPAD - EMBED)))
    pooled = _pooled_sc(inputs.astype(jnp.int32), table128)
    w2 = jnp.pad(W, ((0, EPAD - EMBED), (0, 0)))
    return _dense_tc(pooled, w2, b.reshape(1, NUM_TARGETS))


# speedup vs baseline: 1.0059x; 1.0059x over previous
import jax
import jax.numpy as jnp
from jax import lax
from jax.experimental import pallas as pl
from jax.experimental.pallas import tpu as pltpu
from jax.experimental.pallas import tpu_sc as plsc

BATCH = 4096
SEQ = 200
EMBED = 64
NUM_TARGETS = 3811

NC, NS = 2, 16
NW = NC * NS
ROWS_PER_W = BATCH // NW
CB = 4
NBLK = ROWS_PER_W // CB
LANES = 16
CCH = EMBED // LANES
GCH = SEQ // 2


def _pooling_kernel(idx_hbm, table_hbm, out_hbm, idx_v, rows_v, out_v, sem):
    wid = lax.axis_index("s") * NC + lax.axis_index("c")
    base = wid * ROWS_PER_W

    def block(blk, _):
        b0 = base + blk * CB
        pltpu.sync_copy(idx_hbm.at[pl.ds(b0, CB)], idx_v)
        cps = []
        for i in range(CB):
            for off, ln in ((0, 128), (128, 72)):
                cps.append(
                    pltpu.async_copy(
                        table_hbm.at[idx_v.at[i, pl.ds(off, ln)]],
                        rows_v.at[i, pl.ds(off, ln)],
                        sem,
                    )
                )
        for cp in cps:
            cp.wait()

        def acc_body(r, acc):
            new = []
            for i in range(CB):
                for c in range(CCH):
                    v = acc[i * CCH + c]
                    v = v + rows_v[i, r, pl.ds(LANES * c, LANES)]
                    v = v + rows_v[i, r + GCH, pl.ds(LANES * c, LANES)]
                    new.append(v)
            return tuple(new)

        zeros = tuple(
            jnp.zeros((LANES,), jnp.float32) for _ in range(CB * CCH)
        )
        acc = lax.fori_loop(0, GCH, acc_body, zeros)
        scale = jnp.float32(1.0 / SEQ)
        for i in range(CB):
            for c in range(CCH):
                out_v[i, pl.ds(LANES * c, LANES)] = acc[i * CCH + c] * scale
        pltpu.sync_copy(out_v, out_hbm.at[pl.ds(b0, CB)])
        return ()

    lax.fori_loop(0, NBLK, block, ())


@jax.jit
def _pooled_sc(idx, table):
    mesh = plsc.VectorSubcoreMesh(
        core_axis_name="c", subcore_axis_name="s", num_cores=NC, num_subcores=NS
    )
    return pl.kernel(
        _pooling_kernel,
        out_type=jax.ShapeDtypeStruct((BATCH, EMBED), jnp.float32),
        mesh=mesh,
        compiler_params=pltpu.CompilerParams(use_tc_tiling_on_sc=False),
        scratch_types=[
            pltpu.VMEM((CB, SEQ), jnp.int32),
            pltpu.VMEM((CB, SEQ, EMBED), jnp.float32),
            pltpu.VMEM((CB, EMBED), jnp.float32),
            pltpu.SemaphoreType.DMA,
        ],
    )(idx, table)


def _dense_kernel(x_ref, w_ref, b_ref, o_ref):
    y = jnp.dot(x_ref[...], w_ref[...], preferred_element_type=jnp.float32)
    o_ref[...] = jax.nn.sigmoid(y + b_ref[...])


BM = 512


@jax.jit
def _dense_tc(pooled, w2, b2):
    return pl.pallas_call(
        _dense_kernel,
        grid=(BATCH // BM,),
        in_specs=[
            pl.BlockSpec((BM, EMBED), lambda i: (i, 0)),
            pl.BlockSpec((EMBED, NUM_TARGETS), lambda i: (0, 0)),
            pl.BlockSpec((1, NUM_TARGETS), lambda i: (0, 0)),
        ],
        out_specs=pl.BlockSpec((BM, NUM_TARGETS), lambda i: (i, 0)),
        out_shape=jax.ShapeDtypeStruct((BATCH, NUM_TARGETS), jnp.float32),
    )(pooled, w2, b2)


def kernel(inputs, table, W, b):
    pooled = _pooled_sc(inputs.astype(jnp.int32), table)
    return _dense_tc(pooled, W, b.reshape(1, NUM_TARGETS))
